# Initial kernel scaffold; baseline (speedup 1.0000x reference)
import jax
import jax.numpy as jnp
from jax import lax
from jax.experimental import pallas as pl
from jax.experimental.pallas import tpu as pltpu
from jax.experimental.pallas import tpu_sc as plsc

E = 300
NC, NS = 2, 16
NW = NC * NS
C = 128


def _sc_body(q_idx, d_idx, qt, dt, q_out, d_out, idx_v, rows_v, sem):
    wid = lax.axis_index("s") * NC + lax.axis_index("c")

    def phase(idx_hbm, table, out_hbm):
        per_w = idx_hbm.shape[0] // NW
        base_w = wid * per_w

        @pl.loop(0, per_w // C)
        def _(i):
            base = base_w + i * C
            pltpu.sync_copy(idx_hbm.at[pl.ds(base, C)], idx_v)
            pltpu.async_copy(table.at[idx_v], rows_v, sem).wait()
            pltpu.sync_copy(rows_v, out_hbm.at[pl.ds(base, C)])

    phase(q_idx, qt, q_out)
    phase(d_idx, dt, d_out)


def kernel(query, doc, query_table, doc_table):
    B, Lq = query.shape
    _, Ld = doc.shape
    q_idx = query.reshape(-1).astype(jnp.int32)
    d_idx = doc.reshape(-1).astype(jnp.int32)

    call = pl.kernel(
        _sc_body,
        out_type=(
            jax.ShapeDtypeStruct((q_idx.shape[0], E), jnp.float32),
            jax.ShapeDtypeStruct((d_idx.shape[0], E), jnp.float32),
        ),
        mesh=plsc.VectorSubcoreMesh(
            core_axis_name="c", subcore_axis_name="s",
            num_cores=NC, num_subcores=NS,
        ),
        scratch_types=[
            pltpu.VMEM((C,), jnp.int32),
            pltpu.VMEM((C, E), jnp.float32),
            pltpu.SemaphoreType.DMA,
        ],
    )
    q_rows, d_rows = call(q_idx, d_idx, query_table, doc_table)
    return q_rows.reshape(B, Lq, E), d_rows.reshape(B, Ld, E)


# speedup vs baseline: 1.4842x; 1.4842x over previous
import jax
import jax.numpy as jnp
from jax import lax
from jax.experimental import pallas as pl
from jax.experimental.pallas import tpu as pltpu
from jax.experimental.pallas import tpu_sc as plsc

E = 300
EP = 304
NC, NS = 2, 16
NW = NC * NS
C = 128


def _sc_body(q_idx, d_idx, qt, dt, q_out, d_out, idx_v, rows_v, sem):
    wid = lax.axis_index("s") * NC + lax.axis_index("c")

    def phase(idx_hbm, table, out_hbm):
        per_w = idx_hbm.shape[0] // NW
        base_w = wid * per_w

        @pl.loop(0, per_w // C)
        def _(i):
            base = base_w + i * C
            pltpu.sync_copy(idx_hbm.at[pl.ds(base, C)], idx_v)
            pltpu.async_copy(table.at[idx_v], rows_v, sem).wait()
            pltpu.sync_copy(rows_v, out_hbm.at[pl.ds(base, C)])

    phase(q_idx, qt, q_out)
    phase(d_idx, dt, d_out)


def kernel(query, doc, query_table, doc_table):
    B, Lq = query.shape
    _, Ld = doc.shape
    q_idx = query.reshape(-1).astype(jnp.int32)
    d_idx = doc.reshape(-1).astype(jnp.int32)
    qt = jnp.pad(query_table, ((0, 0), (0, EP - E)))
    dt = jnp.pad(doc_table, ((0, 0), (0, EP - E)))

    call = pl.kernel(
        _sc_body,
        out_type=(
            jax.ShapeDtypeStruct((q_idx.shape[0], EP), jnp.float32),
            jax.ShapeDtypeStruct((d_idx.shape[0], EP), jnp.float32),
        ),
        mesh=plsc.VectorSubcoreMesh(
            core_axis_name="c", subcore_axis_name="s",
            num_cores=NC, num_subcores=NS,
        ),
        scratch_types=[
            pltpu.VMEM((C,), jnp.int32),
            pltpu.VMEM((C, EP), jnp.float32),
            pltpu.SemaphoreType.DMA,
        ],
        compiler_params=pltpu.CompilerParams(use_tc_tiling_on_sc=False),
    )
    q_rows, d_rows = call(q_idx, d_idx, qt, dt)
    return (q_rows[:, :E].reshape(B, Lq, E),
            d_rows[:, :E].reshape(B, Ld, E))


# speedup vs baseline: 2.2961x; 1.5470x over previous
import jax
import jax.numpy as jnp
from jax import lax
from jax.experimental import pallas as pl
from jax.experimental.pallas import tpu as pltpu
from jax.experimental.pallas import tpu_sc as plsc

E = 300
EP = 384
NC, NS = 2, 16
NW = NC * NS
C = 128


def _sc_body(q_idx, d_idx, qt, dt, q_out, d_out, idx_v, rows_v, sem):
    wid = lax.axis_index("s") * NC + lax.axis_index("c")

    def phase(idx_hbm, table, out_hbm):
        per_w = idx_hbm.shape[0] // NW
        base_w = wid * per_w

        @pl.loop(0, per_w // C)
        def _(i):
            base = base_w + i * C
            pltpu.sync_copy(idx_hbm.at[pl.ds(base, C)], idx_v)
            pltpu.async_copy(table.at[idx_v], rows_v, sem).wait()
            pltpu.sync_copy(rows_v, out_hbm.at[pl.ds(base, C)])

    phase(q_idx, qt, q_out)
    phase(d_idx, dt, d_out)


def kernel(query, doc, query_table, doc_table):
    B, Lq = query.shape
    _, Ld = doc.shape
    q_idx = query.reshape(-1).astype(jnp.int32)
    d_idx = doc.reshape(-1).astype(jnp.int32)
    qt = jnp.pad(query_table, ((0, 0), (0, EP - E)))
    dt = jnp.pad(doc_table, ((0, 0), (0, EP - E)))

    call = pl.kernel(
        _sc_body,
        out_type=(
            jax.ShapeDtypeStruct((q_idx.shape[0], EP), jnp.float32),
            jax.ShapeDtypeStruct((d_idx.shape[0], EP), jnp.float32),
        ),
        mesh=plsc.VectorSubcoreMesh(
            core_axis_name="c", subcore_axis_name="s",
            num_cores=NC, num_subcores=NS,
        ),
        scratch_types=[
            pltpu.VMEM((C,), jnp.int32),
            pltpu.VMEM((C, EP), jnp.float32),
            pltpu.SemaphoreType.DMA,
        ],
    )
    q_rows, d_rows = call(q_idx, d_idx, qt, dt)
    return (q_rows[:, :E].reshape(B, Lq, E),
            d_rows[:, :E].reshape(B, Ld, E))


# speedup vs baseline: 2.5976x; 1.1313x over previous
import jax
import jax.numpy as jnp
from jax import lax
from jax.experimental import pallas as pl
from jax.experimental.pallas import tpu as pltpu
from jax.experimental.pallas import tpu_sc as plsc

E = 300
EP = 384
NC, NS = 2, 16
NW = NC * NS
C = 128


def _sc_body(q_idx, d_idx, qt, dt, q_out, d_out,
             idx0, idx1, rows0, rows1, gsem0, gsem1, wsem0, wsem1):
    wid = lax.axis_index("s") * NC + lax.axis_index("c")
    idx = (idx0, idx1)
    rows = (rows0, rows1)
    gsem = (gsem0, gsem1)
    wsem = (wsem0, wsem1)

    def phase(idx_hbm, table, out_hbm):
        per_w = idx_hbm.shape[0] // NW
        n = per_w // C
        base_w = wid * per_w

        def load_and_gather(s, i):
            pltpu.sync_copy(idx_hbm.at[pl.ds(base_w + i * C, C)], idx[s])
            pltpu.async_copy(table.at[idx[s]], rows[s], gsem[s])

        load_and_gather(0, 0)

        @pl.loop(0, n, step=2)
        def _(g):
            for s in (0, 1):
                i = g + s
                nxt = 1 - s

                @pl.when(i + 1 < n)
                def _():
                    @pl.when(i >= 1)
                    def _():
                        pltpu.make_async_copy(
                            rows[nxt], out_hbm.at[pl.ds(0, C)], wsem[nxt]
                        ).wait()
                    load_and_gather(nxt, i + 1)

                pltpu.make_async_copy(table.at[idx[s]], rows[s], gsem[s]).wait()
                pltpu.async_copy(rows[s], out_hbm.at[pl.ds(base_w + i * C, C)],
                                 wsem[s])

        for s in (0, 1):
            pltpu.make_async_copy(
                rows[s], out_hbm.at[pl.ds(0, C)], wsem[s]).wait()

    phase(q_idx, qt, q_out)
    phase(d_idx, dt, d_out)


def kernel(query, doc, query_table, doc_table):
    B, Lq = query.shape
    _, Ld = doc.shape
    q_idx = query.reshape(-1).astype(jnp.int32)
    d_idx = doc.reshape(-1).astype(jnp.int32)
    qt = jnp.pad(query_table, ((0, 0), (0, EP - E)))
    dt = jnp.pad(doc_table, ((0, 0), (0, EP - E)))

    call = pl.kernel(
        _sc_body,
        out_type=(
            jax.ShapeDtypeStruct((q_idx.shape[0], EP), jnp.float32),
            jax.ShapeDtypeStruct((d_idx.shape[0], EP), jnp.float32),
        ),
        mesh=plsc.VectorSubcoreMesh(
            core_axis_name="c", subcore_axis_name="s",
            num_cores=NC, num_subcores=NS,
        ),
        scratch_types=[
            pltpu.VMEM((C,), jnp.int32),
            pltpu.VMEM((C,), jnp.int32),
            pltpu.VMEM((C, EP), jnp.float32),
            pltpu.VMEM((C, EP), jnp.float32),
            pltpu.SemaphoreType.DMA,
            pltpu.SemaphoreType.DMA,
            pltpu.SemaphoreType.DMA,
            pltpu.SemaphoreType.DMA,
        ],
    )
    q_rows, d_rows = call(q_idx, d_idx, qt, dt)
    return (q_rows[:, :E].reshape(B, Lq, E),
            d_rows[:, :E].reshape(B, Ld, E))


# speedup vs baseline: 2.5993x; 1.0007x over previous
import jax
import jax.numpy as jnp
from jax import lax
from jax.experimental import pallas as pl
from jax.experimental.pallas import tpu as pltpu
from jax.experimental.pallas import tpu_sc as plsc

E = 300
EP = 384
NC, NS = 2, 16
NW = NC * NS
C = 128


def _sc_body(q_idx, d_idx, qt, dt, q_out, d_out,
             idx0, idx1, rows0, rows1,
             gsem0, gsem1, wsem0, wsem1, isem0, isem1):
    wid = lax.axis_index("s") * NC + lax.axis_index("c")
    idx = (idx0, idx1)
    rows = (rows0, rows1)
    gsem = (gsem0, gsem1)
    wsem = (wsem0, wsem1)
    isem = (isem0, isem1)

    def phase(idx_hbm, table, out_hbm):
        per_w = idx_hbm.shape[0] // NW
        n = per_w // C
        base_w = wid * per_w

        def idx_load(s, i):
            pltpu.async_copy(idx_hbm.at[pl.ds(base_w + i * C, C)],
                             idx[s], isem[s])

        def idx_wait(s):
            pltpu.make_async_copy(
                idx_hbm.at[pl.ds(0, C)], idx[s], isem[s]).wait()

        idx_load(0, 0)
        idx_load(1, 1)
        idx_wait(0)
        pltpu.async_copy(table.at[idx[0]], rows[0], gsem[0])

        @pl.loop(0, n, step=2)
        def _(g):
            for s in (0, 1):
                i = g + s
                nxt = 1 - s

                @pl.when(i + 1 < n)
                def _():
                    @pl.when(i >= 1)
                    def _():
                        pltpu.make_async_copy(
                            rows[nxt], out_hbm.at[pl.ds(0, C)], wsem[nxt]
                        ).wait()
                    idx_wait(nxt)
                    pltpu.async_copy(table.at[idx[nxt]], rows[nxt], gsem[nxt])

                pltpu.make_async_copy(table.at[idx[s]], rows[s], gsem[s]).wait()
                pltpu.async_copy(rows[s], out_hbm.at[pl.ds(base_w + i * C, C)],
                                 wsem[s])

                @pl.when(i + 2 < n)
                def _():
                    idx_load(s, i + 2)

        for s in (0, 1):
            pltpu.make_async_copy(
                rows[s], out_hbm.at[pl.ds(0, C)], wsem[s]).wait()

    phase(q_idx, qt, q_out)
    phase(d_idx, dt, d_out)


def kernel(query, doc, query_table, doc_table):
    B, Lq = query.shape
    _, Ld = doc.shape
    q_idx = query.reshape(-1).astype(jnp.int32)
    d_idx = doc.reshape(-1).astype(jnp.int32)
    qt = jnp.pad(query_table, ((0, 0), (0, EP - E)))
    dt = jnp.pad(doc_table, ((0, 0), (0, EP - E)))

    call = pl.kernel(
        _sc_body,
        out_type=(
            jax.ShapeDtypeStruct((q_idx.shape[0], EP), jnp.float32),
            jax.ShapeDtypeStruct((d_idx.shape[0], EP), jnp.float32),
        ),
        mesh=plsc.VectorSubcoreMesh(
            core_axis_name="c", subcore_axis_name="s",
            num_cores=NC, num_subcores=NS,
        ),
        scratch_types=[
            pltpu.VMEM((C,), jnp.int32),
            pltpu.VMEM((C,), jnp.int32),
            pltpu.VMEM((C, EP), jnp.float32),
            pltpu.VMEM((C, EP), jnp.float32),
            pltpu.SemaphoreType.DMA,
            pltpu.SemaphoreType.DMA,
            pltpu.SemaphoreType.DMA,
            pltpu.SemaphoreType.DMA,
            pltpu.SemaphoreType.DMA,
            pltpu.SemaphoreType.DMA,
        ],
    )
    q_rows, d_rows = call(q_idx, d_idx, qt, dt)
    return (q_rows[:, :E].reshape(B, Lq, E),
            d_rows[:, :E].reshape(B, Ld, E))


# speedup vs baseline: 2.7534x; 1.0593x over previous
import jax
import jax.numpy as jnp
from jax import lax
from jax.experimental import pallas as pl
from jax.experimental.pallas import tpu as pltpu
from jax.experimental.pallas import tpu_sc as plsc

E = 300
EP = 384
NC, NS = 2, 16
NW = NC * NS
C = 128


def _sc_body(q_idx, d_idx, qt, dt, q_out, d_out,
             idx0, idx1, rows0, rows1,
             gsem0, gsem1, wsem0, wsem1, isem0, isem1):
    wid = lax.axis_index("s") * NC + lax.axis_index("c")
    idx = (idx0, idx1)
    rows = (rows0, rows1)
    gsem = (gsem0, gsem1)
    wsem = (wsem0, wsem1)
    isem = (isem0, isem1)

    def phase(idx_hbm, table, out_hbm):
        per_w = idx_hbm.shape[0] // NW
        n = per_w // C
        base_w = wid * per_w

        def idx_load(s, i):
            pltpu.async_copy(idx_hbm.at[pl.ds(base_w + i * C, C)],
                             idx[s], isem[s])

        def idx_wait(s):
            pltpu.make_async_copy(
                idx_hbm.at[pl.ds(0, C)], idx[s], isem[s]).wait()

        idx_load(0, 0)
        idx_load(1, 1)
        idx_wait(0)
        pltpu.async_copy(table.at[idx[0]], rows[0], gsem[0])

        @pl.loop(0, n, step=2)
        def _(g):
            for s in (0, 1):
                i = g + s
                nxt = 1 - s

                @pl.when(i + 1 < n)
                def _():
                    @pl.when(i >= 1)
                    def _():
                        pltpu.make_async_copy(
                            rows[nxt], out_hbm.at[pl.ds(0, C)], wsem[nxt]
                        ).wait()
                    idx_wait(nxt)
                    pltpu.async_copy(table.at[idx[nxt]], rows[nxt], gsem[nxt])

                pltpu.make_async_copy(table.at[idx[s]], rows[s], gsem[s]).wait()
                pltpu.async_copy(rows[s], out_hbm.at[pl.ds(base_w + i * C, C)],
                                 wsem[s])

                @pl.when(i + 2 < n)
                def _():
                    idx_load(s, i + 2)

        for s in (0, 1):
            pltpu.make_async_copy(
                rows[s], out_hbm.at[pl.ds(0, C)], wsem[s]).wait()

    phase(q_idx, qt, q_out)
    phase(d_idx, dt, d_out)


def kernel(query, doc, query_table, doc_table):
    B, Lq = query.shape
    _, Ld = doc.shape
    V = query_table.shape[0]
    LqP = -(-Lq // 8) * 8
    pad_block = (jnp.arange(B * (LqP - Lq), dtype=jnp.int32) % V).reshape(
        B, LqP - Lq)
    q_idx = jnp.concatenate(
        [query.astype(jnp.int32), pad_block], axis=1).reshape(-1)
    d_idx = doc.reshape(-1).astype(jnp.int32)
    qt = jnp.pad(query_table, ((0, 0), (0, EP - E)))
    dt = jnp.pad(doc_table, ((0, 0), (0, EP - E)))

    call = pl.kernel(
        _sc_body,
        out_type=(
            jax.ShapeDtypeStruct((q_idx.shape[0], EP), jnp.float32),
            jax.ShapeDtypeStruct((d_idx.shape[0], EP), jnp.float32),
        ),
        mesh=plsc.VectorSubcoreMesh(
            core_axis_name="c", subcore_axis_name="s",
            num_cores=NC, num_subcores=NS,
        ),
        scratch_types=[
            pltpu.VMEM((C,), jnp.int32),
            pltpu.VMEM((C,), jnp.int32),
            pltpu.VMEM((C, EP), jnp.float32),
            pltpu.VMEM((C, EP), jnp.float32),
            pltpu.SemaphoreType.DMA,
            pltpu.SemaphoreType.DMA,
            pltpu.SemaphoreType.DMA,
            pltpu.SemaphoreType.DMA,
            pltpu.SemaphoreType.DMA,
            pltpu.SemaphoreType.DMA,
        ],
    )
    q_rows, d_rows = call(q_idx, d_idx, qt, dt)
    return (q_rows.reshape(B, LqP, EP)[:, :Lq, :E],
            d_rows[:, :E].reshape(B, Ld, E))
